# baseline (speedup 1.0000x reference)
import functools

import jax
import jax.numpy as jnp
from jax import lax
from jax.experimental import pallas as pl
from jax.experimental.pallas import tpu as pltpu
from jax.experimental.pallas import tpu_sc as plsc

BATCH = 16384
NUM_ENT = 1000000
NUM_REL = 1000
K = 64
ETA = 5

NC = 2
NS = 16
L = 16
NW = NC * NS
ROWS_W = BATCH // NW
CHUNK = 128
NCHUNK = ROWS_W // CHUNK
NGROUP = CHUNK // L


def _make_sc_call():
    mesh = plsc.VectorSubcoreMesh(core_axis_name="c", subcore_axis_name="s")

    @functools.partial(
        pl.kernel,
        mesh=mesh,
        out_type=(
            jax.ShapeDtypeStruct((BATCH,), jnp.float32),
            jax.ShapeDtypeStruct((BATCH * ETA,), jnp.float32),
        ),
        scratch_types=[
            pltpu.VMEM((CHUNK,), jnp.int32),
            pltpu.VMEM((CHUNK,), jnp.int32),
            pltpu.VMEM((CHUNK,), jnp.int32),
            pltpu.VMEM((CHUNK,), jnp.int32),
            pltpu.VMEM((CHUNK,), jnp.int32),
            pltpu.VMEM((CHUNK,), jnp.int32),
            pltpu.VMEM((CHUNK,), jnp.int32),
            pltpu.VMEM((CHUNK, K), jnp.float32),
            pltpu.VMEM((CHUNK, K), jnp.float32),
            pltpu.VMEM((CHUNK, K), jnp.float32),
            pltpu.VMEM((CHUNK,), jnp.float32),
            pltpu.SemaphoreType.DMA,
            pltpu.SemaphoreType.DMA,
            pltpu.SemaphoreType.DMA,
        ],
        compiler_params=pltpu.CompilerParams(
            needs_layout_passes=False, use_tc_tiling_on_sc=False),
    )
    def sc_call(subj_h, pred_h, obj_h, rand_h, side_h, ent_h, rel_h,
                inp_out, corr_out,
                subj_v, pred_v, obj_v, rand_v, side_v, sidx_v, oidx_v,
                es_v, ep_v, eo_v, score_v, sem_s, sem_p, sem_o):
        wid = lax.axis_index("s") * NC + lax.axis_index("c")
        tri_base = wid * ROWS_W
        lanes = lax.iota(jnp.int32, L)

        def compute_scores(s_ref, p_ref, o_ref):
            cs = pltpu.async_copy(ent_h.at[s_ref], es_v, sem_s)
            cp = pltpu.async_copy(rel_h.at[p_ref], ep_v, sem_p)
            co = pltpu.async_copy(ent_h.at[o_ref], eo_v, sem_o)
            cs.wait()
            cp.wait()
            co.wait()
            def row_group(g, _):
                rows = g * L + lanes
                acc = jnp.zeros((L,), jnp.float32)
                for k in range(K):
                    kv = jnp.zeros((L,), jnp.int32) + k
                    a = plsc.load_gather(es_v, [rows, kv])
                    b = plsc.load_gather(ep_v, [rows, kv])
                    c = plsc.load_gather(eo_v, [rows, kv])
                    acc = acc + a * b * c
                score_v[pl.ds(g * L, L)] = acc
                return 0

            lax.fori_loop(0, NGROUP, row_group, 0, unroll=False)

        def pos_chunk(c, _):
            off = pl.multiple_of(tri_base + c * CHUNK, CHUNK)
            pltpu.sync_copy(subj_h.at[pl.ds(off, CHUNK)], subj_v)
            pltpu.sync_copy(pred_h.at[pl.ds(off, CHUNK)], pred_v)
            pltpu.sync_copy(obj_h.at[pl.ds(off, CHUNK)], obj_v)
            compute_scores(subj_v, pred_v, obj_v)
            pltpu.sync_copy(score_v, inp_out.at[pl.ds(off, CHUNK)])
            return 0

        lax.fori_loop(0, NCHUNK, pos_chunk, 0, unroll=False)

        def corr_chunk(i, _):
            m = i // NCHUNK
            c = i % NCHUNK
            off = pl.multiple_of(tri_base + c * CHUNK, CHUNK)
            doff = pl.multiple_of(m * BATCH + off, CHUNK)
            pltpu.sync_copy(subj_h.at[pl.ds(off, CHUNK)], subj_v)
            pltpu.sync_copy(pred_h.at[pl.ds(off, CHUNK)], pred_v)
            pltpu.sync_copy(obj_h.at[pl.ds(off, CHUNK)], obj_v)
            pltpu.sync_copy(rand_h.at[pl.ds(doff, CHUNK)], rand_v)
            pltpu.sync_copy(side_h.at[pl.ds(doff, CHUNK)], side_v)

            def sel(j, _):
                sl = pl.ds(j * L, L)
                flag = side_v[sl] != 0
                sidx_v[sl] = jnp.where(flag, rand_v[sl], subj_v[sl])
                oidx_v[sl] = jnp.where(flag, obj_v[sl], rand_v[sl])
                return 0

            lax.fori_loop(0, CHUNK // L, sel, 0, unroll=False)
            compute_scores(sidx_v, pred_v, oidx_v)
            pltpu.sync_copy(score_v, corr_out.at[pl.ds(doff, CHUNK)])
            return 0

        lax.fori_loop(0, ETA * NCHUNK, corr_chunk, 0, unroll=False)

    return sc_call


_SC_CALL = _make_sc_call()


def kernel(triples, ent_emb, rel_emb, rand_entities, rand_side):
    subj = jnp.asarray(triples[:, 0], jnp.int32)
    pred = jnp.asarray(triples[:, 1], jnp.int32)
    obj = jnp.asarray(triples[:, 2], jnp.int32)
    side = rand_side.astype(jnp.int32)
    inp_score, corr_score = _SC_CALL(
        subj, pred, obj, rand_entities, side, ent_emb, rel_emb)
    return (inp_score, corr_score)


# speedup vs baseline: 1.0109x; 1.0109x over previous
import functools

import jax
import jax.numpy as jnp
from jax import lax
from jax.experimental import pallas as pl
from jax.experimental.pallas import tpu as pltpu
from jax.experimental.pallas import tpu_sc as plsc

BATCH = 16384
NUM_ENT = 1000000
NUM_REL = 1000
K = 64
ETA = 5

NC = 2
NS = 16
L = 16
NW = NC * NS
ROWS_W = BATCH // NW
CHUNK = 128
NCHUNK = ROWS_W // CHUNK
NGROUP = CHUNK // L


def _make_sc_call():
    mesh = plsc.VectorSubcoreMesh(core_axis_name="c", subcore_axis_name="s")

    @functools.partial(
        pl.kernel,
        mesh=mesh,
        out_type=(
            jax.ShapeDtypeStruct((BATCH,), jnp.float32),
            jax.ShapeDtypeStruct((BATCH * ETA,), jnp.float32),
        ),
        scratch_types=[
            pltpu.VMEM((CHUNK, 3), jnp.int32),
            pltpu.VMEM((CHUNK,), jnp.int32),
            pltpu.VMEM((CHUNK // 4,), jnp.int32),
            pltpu.VMEM((CHUNK,), jnp.int32),
            pltpu.VMEM((CHUNK,), jnp.int32),
            pltpu.VMEM((CHUNK,), jnp.int32),
            pltpu.VMEM((CHUNK, K), jnp.float32),
            pltpu.VMEM((CHUNK, K), jnp.float32),
            pltpu.VMEM((CHUNK, K), jnp.float32),
            pltpu.VMEM((CHUNK,), jnp.float32),
            pltpu.SemaphoreType.DMA,
            pltpu.SemaphoreType.DMA,
            pltpu.SemaphoreType.DMA,
        ],
        compiler_params=pltpu.CompilerParams(
            needs_layout_passes=False, use_tc_tiling_on_sc=False),
    )
    def sc_call(tri_h, rand_h, side_h, ent_h, rel_h,
                inp_out, corr_out,
                tri_v, rand_v, side_v, sidx_v, pidx_v, oidx_v,
                es_v, ep_v, eo_v, score_v, sem_s, sem_p, sem_o):
        wid = lax.axis_index("s") * NC + lax.axis_index("c")
        tri_base = wid * ROWS_W
        lanes = lax.iota(jnp.int32, L)
        col0 = jnp.zeros((L,), jnp.int32)
        col1 = col0 + 1
        col2 = col0 + 2

        def compute_scores():
            cs = pltpu.async_copy(ent_h.at[sidx_v], es_v, sem_s)
            cp = pltpu.async_copy(rel_h.at[pidx_v], ep_v, sem_p)
            co = pltpu.async_copy(ent_h.at[oidx_v], eo_v, sem_o)
            cs.wait()
            cp.wait()
            co.wait()

            def row_group(g, _):
                rows = g * L + lanes
                acc = jnp.zeros((L,), jnp.float32)
                for k in range(K):
                    kv = col0 + k
                    a = plsc.load_gather(es_v, [rows, kv])
                    b = plsc.load_gather(ep_v, [rows, kv])
                    c = plsc.load_gather(eo_v, [rows, kv])
                    acc = acc + a * b * c
                score_v[pl.ds(g * L, L)] = acc
                return 0

            lax.fori_loop(0, NGROUP, row_group, 0, unroll=False)

        def pos_chunk(c, _):
            off = pl.multiple_of(tri_base + c * CHUNK, CHUNK)
            pltpu.sync_copy(tri_h.at[pl.ds(off, CHUNK)], tri_v)

            def split(j, _):
                rows = j * L + lanes
                sl = pl.ds(j * L, L)
                sidx_v[sl] = plsc.load_gather(tri_v, [rows, col0])
                pidx_v[sl] = plsc.load_gather(tri_v, [rows, col1])
                oidx_v[sl] = plsc.load_gather(tri_v, [rows, col2])
                return 0

            lax.fori_loop(0, NGROUP, split, 0, unroll=False)
            compute_scores()
            pltpu.sync_copy(score_v, inp_out.at[pl.ds(off, CHUNK)])
            return 0

        lax.fori_loop(0, NCHUNK, pos_chunk, 0, unroll=False)

        def corr_chunk(i, _):
            m = i // NCHUNK
            c = i % NCHUNK
            off = pl.multiple_of(tri_base + c * CHUNK, CHUNK)
            doff = pl.multiple_of(m * BATCH + off, CHUNK)
            pltpu.sync_copy(tri_h.at[pl.ds(off, CHUNK)], tri_v)
            pltpu.sync_copy(rand_h.at[pl.ds(doff, CHUNK)], rand_v)
            soff = pl.multiple_of(doff // 4, CHUNK // 4)
            pltpu.sync_copy(side_h.at[pl.ds(soff, CHUNK // 4)], side_v)

            def sel(j, _):
                rows = j * L + lanes
                sl = pl.ds(j * L, L)
                s = plsc.load_gather(tri_v, [rows, col0])
                p = plsc.load_gather(tri_v, [rows, col1])
                o = plsc.load_gather(tri_v, [rows, col2])
                r = rand_v[sl]
                words = plsc.load_gather(side_v, [rows >> 2])
                bits = (words >> ((lanes & 3) * 8)) & 1
                flag = bits != 0
                sidx_v[sl] = jnp.where(flag, r, s)
                pidx_v[sl] = p
                oidx_v[sl] = jnp.where(flag, o, r)
                return 0

            lax.fori_loop(0, NGROUP, sel, 0, unroll=False)
            compute_scores()
            pltpu.sync_copy(score_v, corr_out.at[pl.ds(doff, CHUNK)])
            return 0

        lax.fori_loop(0, ETA * NCHUNK, corr_chunk, 0, unroll=False)

    return sc_call


_SC_CALL = _make_sc_call()


def kernel(triples, ent_emb, rel_emb, rand_entities, rand_side):
    side_words = lax.bitcast_convert_type(
        rand_side.reshape(-1, 4).astype(jnp.uint8), jnp.int32)
    inp_score, corr_score = _SC_CALL(
        triples, rand_entities, side_words, ent_emb, rel_emb)
    return (inp_score, corr_score)


# speedup vs baseline: 1.0123x; 1.0014x over previous
import functools

import jax
import jax.numpy as jnp
from jax import lax
from jax.experimental import pallas as pl
from jax.experimental.pallas import tpu as pltpu
from jax.experimental.pallas import tpu_sc as plsc

BATCH = 16384
NUM_ENT = 1000000
NUM_REL = 1000
K = 64
ETA = 5

NC = 2
NS = 16
L = 16
NW = NC * NS
ROWS_W = BATCH // NW
CHUNK = 128
NCHUNK = ROWS_W // CHUNK
NGROUP = CHUNK // L


def _make_sc_call():
    mesh = plsc.VectorSubcoreMesh(core_axis_name="c", subcore_axis_name="s")

    @functools.partial(
        pl.kernel,
        mesh=mesh,
        out_type=(
            jax.ShapeDtypeStruct((BATCH,), jnp.float32),
            jax.ShapeDtypeStruct((BATCH * ETA,), jnp.float32),
        ),
        scratch_types=[
            pltpu.VMEM((CHUNK, 3), jnp.int32),
            pltpu.VMEM((CHUNK,), jnp.int32),
            pltpu.VMEM((CHUNK,), jnp.int32),
            pltpu.VMEM((CHUNK,), jnp.int32),
            pltpu.VMEM((CHUNK,), jnp.int32),
            pltpu.VMEM((CHUNK,), jnp.int32),
            pltpu.VMEM((CHUNK, K), jnp.float32),
            pltpu.VMEM((CHUNK, K), jnp.float32),
            pltpu.VMEM((CHUNK, K), jnp.float32),
            pltpu.VMEM((CHUNK,), jnp.float32),
            pltpu.SemaphoreType.DMA,
            pltpu.SemaphoreType.DMA,
            pltpu.SemaphoreType.DMA,
        ],
        compiler_params=pltpu.CompilerParams(
            needs_layout_passes=False, use_tc_tiling_on_sc=False),
    )
    def sc_call(tri_h, rand_h, side_h, ent_h, rel_h,
                inp_out, corr_out,
                tri_v, rand_v, side_v, sidx_v, pidx_v, oidx_v,
                es_v, ep_v, eo_v, score_v, sem_s, sem_p, sem_o):
        wid = lax.axis_index("s") * NC + lax.axis_index("c")
        tri_base = wid * ROWS_W
        lanes = lax.iota(jnp.int32, L)
        col0 = jnp.zeros((L,), jnp.int32)
        col1 = col0 + 1
        col2 = col0 + 2

        def compute_scores():
            cs = pltpu.async_copy(ent_h.at[sidx_v], es_v, sem_s)
            cp = pltpu.async_copy(rel_h.at[pidx_v], ep_v, sem_p)
            co = pltpu.async_copy(ent_h.at[oidx_v], eo_v, sem_o)
            cs.wait()
            cp.wait()
            co.wait()

            def row_group(g, _):
                rows = g * L + lanes
                acc = jnp.zeros((L,), jnp.float32)
                for k in range(K):
                    kv = col0 + k
                    a = plsc.load_gather(es_v, [rows, kv])
                    b = plsc.load_gather(ep_v, [rows, kv])
                    c = plsc.load_gather(eo_v, [rows, kv])
                    acc = acc + a * b * c
                score_v[pl.ds(g * L, L)] = acc
                return 0

            lax.fori_loop(0, NGROUP, row_group, 0, unroll=False)

        def pos_chunk(c, _):
            off = pl.multiple_of(tri_base + c * CHUNK, CHUNK)
            pltpu.sync_copy(tri_h.at[pl.ds(off, CHUNK)], tri_v)

            def split(j, _):
                rows = j * L + lanes
                sl = pl.ds(j * L, L)
                sidx_v[sl] = plsc.load_gather(tri_v, [rows, col0])
                pidx_v[sl] = plsc.load_gather(tri_v, [rows, col1])
                oidx_v[sl] = plsc.load_gather(tri_v, [rows, col2])
                return 0

            lax.fori_loop(0, NGROUP, split, 0, unroll=False)
            compute_scores()
            pltpu.sync_copy(score_v, inp_out.at[pl.ds(off, CHUNK)])
            return 0

        lax.fori_loop(0, NCHUNK, pos_chunk, 0, unroll=False)

        def corr_chunk(i, _):
            m = i // NCHUNK
            c = i % NCHUNK
            off = pl.multiple_of(tri_base + c * CHUNK, CHUNK)
            doff = pl.multiple_of(m * BATCH + off, CHUNK)
            pltpu.sync_copy(tri_h.at[pl.ds(off, CHUNK)], tri_v)
            pltpu.sync_copy(rand_h.at[pl.ds(doff, CHUNK)], rand_v)
            pltpu.sync_copy(side_h.at[pl.ds(doff, CHUNK)], side_v)

            def sel(j, _):
                rows = j * L + lanes
                sl = pl.ds(j * L, L)
                s = plsc.load_gather(tri_v, [rows, col0])
                p = plsc.load_gather(tri_v, [rows, col1])
                o = plsc.load_gather(tri_v, [rows, col2])
                r = rand_v[sl]
                flag = side_v[sl] != 0
                sidx_v[sl] = jnp.where(flag, r, s)
                pidx_v[sl] = p
                oidx_v[sl] = jnp.where(flag, o, r)
                return 0

            lax.fori_loop(0, NGROUP, sel, 0, unroll=False)
            compute_scores()
            pltpu.sync_copy(score_v, corr_out.at[pl.ds(doff, CHUNK)])
            return 0

        lax.fori_loop(0, ETA * NCHUNK, corr_chunk, 0, unroll=False)

    return sc_call


_SC_CALL = _make_sc_call()


def kernel(triples, ent_emb, rel_emb, rand_entities, rand_side):
    side = rand_side.astype(jnp.int32)
    inp_score, corr_score = _SC_CALL(
        triples, rand_entities, side, ent_emb, rel_emb)
    return (inp_score, corr_score)


# speedup vs baseline: 1.0471x; 1.0343x over previous
import functools

import jax
import jax.numpy as jnp
from jax import lax
from jax.experimental import pallas as pl
from jax.experimental.pallas import tpu as pltpu
from jax.experimental.pallas import tpu_sc as plsc

BATCH = 16384
NUM_ENT = 1000000
NUM_REL = 1000
K = 64
ETA = 5

NC = 2
NS = 16
L = 16
NW = NC * NS
ROWS_W = BATCH // NW
CHUNK = 128
NCHUNK = ROWS_W // CHUNK
NGROUP = CHUNK // L
KP = 128


def _make_sc_call():
    mesh = plsc.VectorSubcoreMesh(core_axis_name="c", subcore_axis_name="s")

    @functools.partial(
        pl.kernel,
        mesh=mesh,
        out_type=(
            jax.ShapeDtypeStruct((BATCH,), jnp.float32),
            jax.ShapeDtypeStruct((BATCH * ETA,), jnp.float32),
        ),
        scratch_types=[
            pltpu.VMEM((CHUNK, 3), jnp.int32),
            pltpu.VMEM((CHUNK,), jnp.int32),
            pltpu.VMEM((CHUNK,), jnp.int32),
            pltpu.VMEM((CHUNK,), jnp.int32),
            pltpu.VMEM((CHUNK,), jnp.int32),
            pltpu.VMEM((CHUNK,), jnp.int32),
            pltpu.VMEM((CHUNK, KP), jnp.float32),
            pltpu.VMEM((CHUNK, KP), jnp.float32),
            pltpu.VMEM((CHUNK, KP), jnp.float32),
            pltpu.VMEM((CHUNK,), jnp.float32),
            pltpu.SemaphoreType.DMA,
            pltpu.SemaphoreType.DMA,
            pltpu.SemaphoreType.DMA,
        ],
        compiler_params=pltpu.CompilerParams(
            needs_layout_passes=False, use_tc_tiling_on_sc=False),
    )
    def sc_call(tri_h, rand_h, side_h, ent_h, rel_h,
                inp_out, corr_out,
                tri_v, rand_v, side_v, sidx_v, pidx_v, oidx_v,
                es_v, ep_v, eo_v, score_v, sem_s, sem_p, sem_o):
        wid = lax.axis_index("s") * NC + lax.axis_index("c")
        tri_base = wid * ROWS_W
        lanes = lax.iota(jnp.int32, L)
        col0 = jnp.zeros((L,), jnp.int32)
        col1 = col0 + 1
        col2 = col0 + 2

        def compute_scores():
            cs = pltpu.async_copy(ent_h.at[sidx_v], es_v, sem_s)
            cp = pltpu.async_copy(rel_h.at[pidx_v], ep_v, sem_p)
            co = pltpu.async_copy(ent_h.at[oidx_v], eo_v, sem_o)
            cs.wait()
            cp.wait()
            co.wait()

            def row_group(g, _):
                rows = g * L + lanes
                acc = jnp.zeros((L,), jnp.float32)
                for k in range(K):
                    kv = col0 + k
                    a = plsc.load_gather(es_v, [rows, kv])
                    b = plsc.load_gather(ep_v, [rows, kv])
                    c = plsc.load_gather(eo_v, [rows, kv])
                    acc = acc + a * b * c
                score_v[pl.ds(g * L, L)] = acc
                return 0

            lax.fori_loop(0, NGROUP, row_group, 0, unroll=False)

        def pos_chunk(c, _):
            off = pl.multiple_of(tri_base + c * CHUNK, CHUNK)
            pltpu.sync_copy(tri_h.at[pl.ds(off, CHUNK)], tri_v)

            def split(j, _):
                rows = j * L + lanes
                sl = pl.ds(j * L, L)
                sidx_v[sl] = plsc.load_gather(tri_v, [rows, col0])
                pidx_v[sl] = plsc.load_gather(tri_v, [rows, col1])
                oidx_v[sl] = plsc.load_gather(tri_v, [rows, col2])
                return 0

            lax.fori_loop(0, NGROUP, split, 0, unroll=False)
            compute_scores()
            pltpu.sync_copy(score_v, inp_out.at[pl.ds(off, CHUNK)])
            return 0

        lax.fori_loop(0, NCHUNK, pos_chunk, 0, unroll=False)

        def corr_chunk(i, _):
            m = i // NCHUNK
            c = i % NCHUNK
            off = pl.multiple_of(tri_base + c * CHUNK, CHUNK)
            doff = pl.multiple_of(m * BATCH + off, CHUNK)
            pltpu.sync_copy(tri_h.at[pl.ds(off, CHUNK)], tri_v)
            pltpu.sync_copy(rand_h.at[pl.ds(doff, CHUNK)], rand_v)
            pltpu.sync_copy(side_h.at[pl.ds(doff, CHUNK)], side_v)

            def sel(j, _):
                rows = j * L + lanes
                sl = pl.ds(j * L, L)
                s = plsc.load_gather(tri_v, [rows, col0])
                p = plsc.load_gather(tri_v, [rows, col1])
                o = plsc.load_gather(tri_v, [rows, col2])
                r = rand_v[sl]
                flag = side_v[sl] != 0
                sidx_v[sl] = jnp.where(flag, r, s)
                pidx_v[sl] = p
                oidx_v[sl] = jnp.where(flag, o, r)
                return 0

            lax.fori_loop(0, NGROUP, sel, 0, unroll=False)
            compute_scores()
            pltpu.sync_copy(score_v, corr_out.at[pl.ds(doff, CHUNK)])
            return 0

        lax.fori_loop(0, ETA * NCHUNK, corr_chunk, 0, unroll=False)

    return sc_call


_SC_CALL = _make_sc_call()


def kernel(triples, ent_emb, rel_emb, rand_entities, rand_side):
    side = rand_side.astype(jnp.int32)
    ent_pad = jnp.pad(ent_emb, ((0, 0), (0, KP - K)))
    rel_pad = jnp.pad(rel_emb, ((0, 0), (0, KP - K)))
    inp_score, corr_score = _SC_CALL(
        triples, rand_entities, side, ent_pad, rel_pad)
    return (inp_score, corr_score)


# speedup vs baseline: 1.1520x; 1.1002x over previous
import functools

import jax
import jax.numpy as jnp
from jax import lax
from jax.experimental import pallas as pl
from jax.experimental.pallas import tpu as pltpu
from jax.experimental.pallas import tpu_sc as plsc

BATCH = 16384
NUM_ENT = 1000000
NUM_REL = 1000
K = 64
ETA = 5

NC = 2
NS = 16
L = 16
NW = NC * NS
ROWS_W = BATCH // NW
CHUNK = 128
NGROUP = CHUNK // L
KP = 128
NG = ETA + 1
TOT = NG * ROWS_W
NSTEP = TOT // CHUNK
CORR_W = ETA * ROWS_W


def _make_sc_call():
    mesh = plsc.VectorSubcoreMesh(core_axis_name="c", subcore_axis_name="s")

    @functools.partial(
        pl.kernel,
        mesh=mesh,
        out_type=(
            jax.ShapeDtypeStruct((BATCH,), jnp.float32),
            jax.ShapeDtypeStruct((BATCH * ETA,), jnp.float32),
        ),
        scratch_types=[
            pltpu.VMEM((ROWS_W, 3), jnp.int32),
            pltpu.VMEM((CORR_W,), jnp.int32),
            pltpu.VMEM((CORR_W,), jnp.int32),
            pltpu.VMEM((TOT,), jnp.int32),
            pltpu.VMEM((TOT,), jnp.int32),
            pltpu.VMEM((TOT,), jnp.int32),
            pltpu.VMEM((TOT,), jnp.float32),
            pltpu.VMEM((CHUNK, KP), jnp.float32),
            pltpu.VMEM((CHUNK, KP), jnp.float32),
            pltpu.VMEM((CHUNK, KP), jnp.float32),
            pltpu.VMEM((CHUNK, KP), jnp.float32),
            pltpu.VMEM((CHUNK, KP), jnp.float32),
            pltpu.VMEM((CHUNK, KP), jnp.float32),
            pltpu.SemaphoreType.DMA,
            pltpu.SemaphoreType.DMA,
            pltpu.SemaphoreType.DMA,
            pltpu.SemaphoreType.DMA,
            pltpu.SemaphoreType.DMA,
            pltpu.SemaphoreType.DMA,
        ],
        compiler_params=pltpu.CompilerParams(
            needs_layout_passes=False, use_tc_tiling_on_sc=False),
    )
    def sc_call(tri_h, rand_h, side_h, ent_h, rel_h,
                inp_out, corr_out,
                tri_all, rand_all, side_all, sidx, pidx, oidx, score_all,
                es0, es1, ep0, ep1, eo0, eo1,
                ss0, ss1, sp0, sp1, so0, so1):
        wid = lax.axis_index("s") * NC + lax.axis_index("c")
        tri_base = pl.multiple_of(wid * ROWS_W, ROWS_W)
        corr_base = pl.multiple_of(wid * ROWS_W, ROWS_W)
        lanes = lax.iota(jnp.int32, L)
        col0 = jnp.zeros((L,), jnp.int32)
        col1 = col0 + 1
        col2 = col0 + 2

        es = (es0, es1)
        ep = (ep0, ep1)
        eo = (eo0, eo1)
        ss = (ss0, ss1)
        sp = (sp0, sp1)
        so = (so0, so1)

        pltpu.sync_copy(tri_h.at[pl.ds(tri_base, ROWS_W)], tri_all)
        for m in range(ETA):
            doff = pl.multiple_of(m * BATCH + corr_base, ROWS_W)
            dsl = pl.ds(m * ROWS_W, ROWS_W)
            pltpu.sync_copy(rand_h.at[pl.ds(doff, ROWS_W)], rand_all.at[dsl])
            pltpu.sync_copy(side_h.at[pl.ds(doff, ROWS_W)], side_all.at[dsl])

        def build(i, _):
            g = i // (ROWS_W // L)
            ltr = (i * L - g * ROWS_W) + lanes
            s = plsc.load_gather(tri_all, [ltr, col0])
            p = plsc.load_gather(tri_all, [ltr, col1])
            o = plsc.load_gather(tri_all, [ltr, col2])
            co = jnp.maximum(i * L - ROWS_W, 0)
            r = rand_all[pl.ds(co, L)]
            f = side_all[pl.ds(co, L)] != 0
            gv = (col0 + g) > 0
            fx = f & gv
            fx2 = f | (~gv)
            sl = pl.ds(i * L, L)
            sidx[sl] = jnp.where(fx, r, s)
            pidx[sl] = p
            oidx[sl] = jnp.where(fx2, o, r)
            return 0

        lax.fori_loop(0, TOT // L, build, 0, unroll=False)

        def fire(t, b):
            off = pl.multiple_of(t * CHUNK, CHUNK)
            pltpu.async_copy(ent_h.at[sidx.at[pl.ds(off, CHUNK)]], es[b], ss[b])
            pltpu.async_copy(rel_h.at[pidx.at[pl.ds(off, CHUNK)]], ep[b], sp[b])
            pltpu.async_copy(ent_h.at[oidx.at[pl.ds(off, CHUNK)]], eo[b], so[b])

        def compute(t, b):
            off = pl.multiple_of(t * CHUNK, CHUNK)
            pltpu.make_async_copy(
                ent_h.at[sidx.at[pl.ds(off, CHUNK)]], es[b], ss[b]).wait()
            pltpu.make_async_copy(
                rel_h.at[pidx.at[pl.ds(off, CHUNK)]], ep[b], sp[b]).wait()
            pltpu.make_async_copy(
                ent_h.at[oidx.at[pl.ds(off, CHUNK)]], eo[b], so[b]).wait()

            def row_group(g, _):
                rows = g * L + lanes
                acc = jnp.zeros((L,), jnp.float32)
                for k in range(K):
                    kv = col0 + k
                    a = plsc.load_gather(es[b], [rows, kv])
                    bb = plsc.load_gather(ep[b], [rows, kv])
                    c = plsc.load_gather(eo[b], [rows, kv])
                    acc = acc + a * bb * c
                score_all[pl.ds(off + g * L, L)] = acc
                return 0

            lax.fori_loop(0, NGROUP, row_group, 0, unroll=False)

        fire(0, 0)

        def step(s2, _):
            t0 = s2 * 2
            fire(t0 + 1, 1)
            compute(t0, 0)

            @pl.when(s2 < NSTEP // 2 - 1)
            def _():
                fire(t0 + 2, 0)

            compute(t0 + 1, 1)
            return 0

        lax.fori_loop(0, NSTEP // 2, step, 0, unroll=False)

        pltpu.sync_copy(score_all.at[pl.ds(0, ROWS_W)],
                        inp_out.at[pl.ds(tri_base, ROWS_W)])
        for m in range(ETA):
            doff = pl.multiple_of(m * BATCH + corr_base, ROWS_W)
            pltpu.sync_copy(score_all.at[pl.ds((m + 1) * ROWS_W, ROWS_W)],
                            corr_out.at[pl.ds(doff, ROWS_W)])

    return sc_call


_SC_CALL = _make_sc_call()


def kernel(triples, ent_emb, rel_emb, rand_entities, rand_side):
    side = rand_side.astype(jnp.int32)
    ent_pad = jnp.pad(ent_emb, ((0, 0), (0, KP - K)))
    rel_pad = jnp.pad(rel_emb, ((0, 0), (0, KP - K)))
    inp_score, corr_score = _SC_CALL(
        triples, rand_entities, side, ent_pad, rel_pad)
    return (inp_score, corr_score)


# speedup vs baseline: 1.5568x; 1.3513x over previous
import functools

import jax
import jax.numpy as jnp
from jax import lax
from jax.experimental import pallas as pl
from jax.experimental.pallas import tpu as pltpu
from jax.experimental.pallas import tpu_sc as plsc

BATCH = 16384
NUM_ENT = 1000000
NUM_REL = 1000
K = 64
ETA = 5

NC = 2
NS = 16
L = 16
NW = NC * NS
ROWS_W = BATCH // NW
CHUNK = 128
NGROUP = CHUNK // L
KP = 128
NG = ETA + 1
TOT = NG * ROWS_W
NSTEP = TOT // CHUNK
CORR_W = ETA * ROWS_W


def _make_sc_call():
    mesh = plsc.VectorSubcoreMesh(core_axis_name="c", subcore_axis_name="s")

    @functools.partial(
        pl.kernel,
        mesh=mesh,
        out_type=(
            jax.ShapeDtypeStruct((BATCH,), jnp.float32),
            jax.ShapeDtypeStruct((BATCH * ETA,), jnp.float32),
        ),
        scratch_types=[
            pltpu.VMEM((ROWS_W, 3), jnp.int32),
            pltpu.VMEM((CORR_W,), jnp.int32),
            pltpu.VMEM((CORR_W,), jnp.int32),
            pltpu.VMEM((TOT,), jnp.int32),
            pltpu.VMEM((TOT,), jnp.int32),
            pltpu.VMEM((TOT,), jnp.int32),
            pltpu.VMEM((TOT,), jnp.float32),
            pltpu.VMEM((CHUNK, KP), jnp.float32),
            pltpu.VMEM((CHUNK, KP), jnp.float32),
            pltpu.VMEM((CHUNK, KP), jnp.float32),
            pltpu.VMEM((CHUNK, KP), jnp.float32),
            pltpu.VMEM((CHUNK, KP), jnp.float32),
            pltpu.VMEM((CHUNK, KP), jnp.float32),
            pltpu.SemaphoreType.DMA,
            pltpu.SemaphoreType.DMA,
            pltpu.SemaphoreType.DMA,
            pltpu.SemaphoreType.DMA,
            pltpu.SemaphoreType.DMA,
            pltpu.SemaphoreType.DMA,
        ],
        compiler_params=pltpu.CompilerParams(
            needs_layout_passes=False, use_tc_tiling_on_sc=False),
    )
    def sc_call(tri_h, rand_h, side_h, ent_h, rel_h,
                inp_out, corr_out,
                tri_all, rand_all, side_all, sidx, pidx, oidx, score_all,
                es0, es1, ep0, ep1, eo0, eo1,
                ss0, ss1, sp0, sp1, so0, so1):
        wid = lax.axis_index("s") * NC + lax.axis_index("c")
        tri_base = pl.multiple_of(wid * ROWS_W, ROWS_W)
        corr_base = pl.multiple_of(wid * ROWS_W, ROWS_W)
        lanes = lax.iota(jnp.int32, L)
        col0 = jnp.zeros((L,), jnp.int32)
        col1 = col0 + 1
        col2 = col0 + 2

        es = (es0, es1)
        ep = (ep0, ep1)
        eo = (eo0, eo1)
        ss = (ss0, ss1)
        sp = (sp0, sp1)
        so = (so0, so1)

        pltpu.sync_copy(tri_h.at[pl.ds(tri_base, ROWS_W)], tri_all)
        for m in range(ETA):
            doff = pl.multiple_of(m * BATCH + corr_base, ROWS_W)
            dsl = pl.ds(m * ROWS_W, ROWS_W)
            pltpu.sync_copy(rand_h.at[pl.ds(doff, ROWS_W)], rand_all.at[dsl])
            pltpu.sync_copy(side_h.at[pl.ds(doff, ROWS_W)], side_all.at[dsl])

        def build(i, _):
            g = i // (ROWS_W // L)
            ltr = (i * L - g * ROWS_W) + lanes
            s = plsc.load_gather(tri_all, [ltr, col0])
            p = plsc.load_gather(tri_all, [ltr, col1])
            o = plsc.load_gather(tri_all, [ltr, col2])
            co = jnp.maximum(i * L - ROWS_W, 0)
            r = rand_all[pl.ds(co, L)]
            f = side_all[pl.ds(co, L)] != 0
            gv = (col0 + g) > 0
            fx = f & gv
            fx2 = f | (~gv)
            sl = pl.ds(i * L, L)
            sidx[sl] = jnp.where(fx, r, s)
            pidx[sl] = p
            oidx[sl] = jnp.where(fx2, o, r)
            return 0

        lax.fori_loop(0, TOT // L, build, 0, unroll=False)

        def fire(t, b):
            off = pl.multiple_of(t * CHUNK, CHUNK)
            pltpu.async_copy(ent_h.at[sidx.at[pl.ds(off, CHUNK)]], es[b], ss[b])
            pltpu.async_copy(rel_h.at[pidx.at[pl.ds(off, CHUNK)]], ep[b], sp[b])
            pltpu.async_copy(ent_h.at[oidx.at[pl.ds(off, CHUNK)]], eo[b], so[b])

        def compute(t, b):
            off = pl.multiple_of(t * CHUNK, CHUNK)
            pltpu.make_async_copy(
                ent_h.at[sidx.at[pl.ds(off, CHUNK)]], es[b], ss[b]).wait()
            pltpu.make_async_copy(
                rel_h.at[pidx.at[pl.ds(off, CHUNK)]], ep[b], sp[b]).wait()
            pltpu.make_async_copy(
                ent_h.at[oidx.at[pl.ds(off, CHUNK)]], eo[b], so[b]).wait()

            def row_group(g, _):
                rows = g * L + lanes
                acc = jnp.zeros((L,), jnp.float32)
                for k in range(K):
                    kv = (lanes + k) & (K - 1)
                    a = plsc.load_gather(es[b], [rows, kv])
                    bb = plsc.load_gather(ep[b], [rows, kv])
                    c = plsc.load_gather(eo[b], [rows, kv])
                    acc = acc + a * bb * c
                score_all[pl.ds(off + g * L, L)] = acc
                return 0

            lax.fori_loop(0, NGROUP, row_group, 0, unroll=False)

        fire(0, 0)

        def step(s2, _):
            t0 = s2 * 2
            fire(t0 + 1, 1)
            compute(t0, 0)

            @pl.when(s2 < NSTEP // 2 - 1)
            def _():
                fire(t0 + 2, 0)

            compute(t0 + 1, 1)
            return 0

        lax.fori_loop(0, NSTEP // 2, step, 0, unroll=False)

        pltpu.sync_copy(score_all.at[pl.ds(0, ROWS_W)],
                        inp_out.at[pl.ds(tri_base, ROWS_W)])
        for m in range(ETA):
            doff = pl.multiple_of(m * BATCH + corr_base, ROWS_W)
            pltpu.sync_copy(score_all.at[pl.ds((m + 1) * ROWS_W, ROWS_W)],
                            corr_out.at[pl.ds(doff, ROWS_W)])

    return sc_call


_SC_CALL = _make_sc_call()


def kernel(triples, ent_emb, rel_emb, rand_entities, rand_side):
    side = rand_side.astype(jnp.int32)
    ent_pad = jnp.pad(ent_emb, ((0, 0), (0, KP - K)))
    rel_pad = jnp.pad(rel_emb, ((0, 0), (0, KP - K)))
    inp_score, corr_score = _SC_CALL(
        triples, rand_entities, side, ent_pad, rel_pad)
    return (inp_score, corr_score)
